# baseline (speedup 1.0000x reference)
import functools

import jax
import jax.numpy as jnp
from jax import lax
from jax.experimental import pallas as pl
from jax.experimental.pallas import tpu as pltpu
from jax.experimental.pallas import tpu_sc as plsc

BATCH = 16384
D = 16
REG = 0.001
NC, NS = 2, 16
NW = NC * NS
RANGE = 32768
NSUB = 16
SUBW = RANGE // NSUB
MCAP = 704
SCAP = 96
SGRP = SCAP // 16
SENT = jnp.int32(0x7FFFFFF0)
OUTN = BATCH * D + 256
ROWS = BATCH // 8


def _sc_gather_body(ps_hbm, qs_hbm, rs_hbm, pt, qt, rt,
                    po, qo, ro,
                    idxall, mval, mpos, sva, spa, svb, spb,
                    flga, flgb, cache, stage, drn,
                    semA, semB, semW):
    wid = lax.axis_index("s") * NC + lax.axis_index("c")
    iota16 = lax.iota(jnp.int32, 16)
    ones16 = jnp.ones((16,), jnp.int32)

    for tab, src_idx, out1d in ((pt, ps_hbm, po), (qt, qs_hbm, qo),
                                (rt, rs_hbm, ro)):
        pltpu.sync_copy(src_idx, idxall)

        for g in range(MCAP // 16):
            mval[pl.ds(g * 16, 16)] = lax.broadcast_in_dim(SENT, (16,), ())

        def scan_body(g, ptr, tab=tab):
            iv = idxall[pl.ds(g * 16, 16)]
            mk = (iv >> 15) == wid
            mk32 = jnp.where(mk, ones16, 0)
            cs = plsc.cumsum(mk32)
            rank = ptr + cs - mk32
            rank = jnp.minimum(rank, MCAP - 1)
            plsc.store_scatter(mval, (rank,), iv, mask=mk)
            plsc.store_scatter(mpos, (rank,), iota16 + g * 16, mask=mk)
            return jnp.minimum(ptr + cs[15], MCAP - 16)

        lax.fori_loop(0, BATCH // 16, scan_body, jnp.int32(0))

        def filt(s, slist, plist, tab=tab):
            for g in range(SGRP):
                slist[pl.ds(g * 16, 16)] = lax.broadcast_in_dim(
                    SENT, (16,), ())
            tag = wid * NSUB + s

            def fb(g, ptr):
                mv = mval[pl.ds(g * 16, 16)]
                pv = mpos[pl.ds(g * 16, 16)]
                mk = (mv >> 11) == tag
                mk32 = jnp.where(mk, ones16, 0)
                cs = plsc.cumsum(mk32)
                rank = jnp.minimum(ptr + cs - mk32, SCAP - 1)
                plsc.store_scatter(slist, (rank,), mv, mask=mk)
                plsc.store_scatter(plist, (rank,), pv, mask=mk)
                return jnp.minimum(ptr + cs[15], SCAP - 16)

            lax.fori_loop(0, MCAP // 16, fb, jnp.int32(0))

        def mkflags(slist, flg):
            flg[...] = jnp.zeros((16,), jnp.int32)
            for g in range(SGRP):
                sv = slist[pl.ds(g * 16, 16)]
                mk = sv < 1000000
                wv = (sv >> 7) & 15
                plsc.store_scatter(flg, (wv,), ones16, mask=mk)

        def fire_windows(s, half, flg, sem, tab=tab):
            fl = flg[...]
            base0 = wid * RANGE + s * SUBW
            for k in range(16):
                @pl.when(fl[k] > 0)
                def _(k=k):
                    cb = pl.multiple_of(base0 + k * 128, 128)
                    pltpu.async_copy(tab.at[:, pl.ds(cb, 128)],
                                     cache.at[half * 16 + k], sem)

        def drain_windows(flg, sem, tab=tab):
            fl = flg[...]
            for k in range(16):
                @pl.when(fl[k] > 0)
                def _(k=k):
                    pltpu.make_async_copy(
                        tab.at[:, pl.ds(0, 128)], cache.at[k], sem).wait()

        def select(half, slist, plist, out1d=out1d):
            for g in range(SGRP):
                sv = slist[pl.ds(g * 16, 16)]
                pv = plist[pl.ds(g * 16, 16)]
                mk = sv < 1000000
                slots = half * 16 + ((sv >> 7) & 15)
                lanes = sv & 127
                offs = jnp.where(mk, pv * D, BATCH * D)
                for k in range(16):
                    colv = lax.broadcast_in_dim(lanes[k], (16,), ())
                    slotv = lax.broadcast_in_dim(slots[k], (16,), ())
                    vals = plsc.load_gather(cache, (slotv, iota16, colv))
                    stage[pl.ds((g * 16 + k) * 16, 16)] = vals
                    off = pl.multiple_of(offs[k], 16)
                    pltpu.async_copy(
                        stage.at[pl.ds((g * 16 + k) * 16, 16)],
                        out1d.at[pl.ds(off, 16)], semW)
            pltpu.make_async_copy(out1d.at[pl.ds(0, SCAP * 16)], drn,
                                  semW).wait()

        filt(0, sva, spa)
        mkflags(sva, flga)
        fire_windows(0, 0, flga, semA)

        def sub_body(i, carry, tab=tab):
            s0 = i * 2
            filt(s0 + 1, svb, spb)
            mkflags(svb, flgb)
            fire_windows(s0 + 1, 1, flgb, semB)
            drain_windows(flga, semA)
            select(0, sva, spa)

            @pl.when(i + 1 < NSUB // 2)
            def _():
                filt(s0 + 2, sva, spa)
                mkflags(sva, flga)
                fire_windows(s0 + 2, 0, flga, semA)
            drain_windows(flgb, semB)
            select(1, svb, spb)
            return carry

        lax.fori_loop(0, NSUB // 2, sub_body, 0)


@functools.cache
def _sc_gather():
    mesh = plsc.VectorSubcoreMesh(
        core_axis_name="c", subcore_axis_name="s",
        num_cores=NC, num_subcores=NS)
    return pl.kernel(
        _sc_gather_body,
        out_type=[jax.ShapeDtypeStruct((OUTN,), jnp.float32)] * 3,
        mesh=mesh,
        compiler_params=pltpu.CompilerParams(
            needs_layout_passes=False, disable_bounds_checks=True),
        scratch_types=[
            pltpu.VMEM((BATCH,), jnp.int32),
            pltpu.VMEM((MCAP,), jnp.int32),
            pltpu.VMEM((MCAP,), jnp.int32),
            pltpu.VMEM((SCAP,), jnp.int32),
            pltpu.VMEM((SCAP,), jnp.int32),
            pltpu.VMEM((SCAP,), jnp.int32),
            pltpu.VMEM((SCAP,), jnp.int32),
            pltpu.VMEM((16,), jnp.int32),
            pltpu.VMEM((16,), jnp.int32),
            pltpu.VMEM((32, 16, 128), jnp.float32),
            pltpu.VMEM((SCAP * 16,), jnp.float32),
            pltpu.VMEM((SCAP * 16,), jnp.float32),
            pltpu.SemaphoreType.DMA,
            pltpu.SemaphoreType.DMA,
            pltpu.SemaphoreType.DMA,
        ],
    )


def _tc_tail_body(pe_ref, qe_ref, re_ref, ww_ref, wb_ref, fcw_ref,
                  inf_ref, regs_ref):
    ww = ww_ref[...]
    wn = jnp.sqrt(jnp.sum(ww * ww, axis=1, keepdims=True))
    wc = ww / jnp.maximum(wn, 1.0)
    fc = fcw_ref[...]
    fn = jnp.sqrt(jnp.sum(fc * fc, axis=1, keepdims=True))
    fcc = fc / jnp.maximum(fn, 1.0)
    pe = pe_ref[...]
    qe = qe_ref[...]
    re = re_ref[...]
    gmf = pe * qe * re
    ri = lax.broadcasted_iota(jnp.int32, (128, 128), 0) // D
    ci = lax.broadcasted_iota(jnp.int32, (128, 128), 1) // D
    blk = ri == ci
    zero = jnp.zeros((128, 128), jnp.float32)
    mp = jnp.where(blk, jnp.tile(wc[:, :D].T, (8, 8)), zero)
    mq = jnp.where(blk, jnp.tile(wc[:, D:2 * D].T, (8, 8)), zero)
    mr = jnp.where(blk, jnp.tile(wc[:, 2 * D:].T, (8, 8)), zero)
    wb_t = jnp.tile(wb_ref[...], (1, 8))
    mlp = (jnp.dot(pe, mp, preferred_element_type=jnp.float32)
           + jnp.dot(qe, mq, preferred_element_type=jnp.float32)
           + jnp.dot(re, mr, preferred_element_type=jnp.float32)
           + wb_t)
    act = jnp.maximum(gmf + mlp, 0.0)
    fi = lax.broadcasted_iota(jnp.int32, (128, 8), 0) // D
    fj = lax.broadcasted_iota(jnp.int32, (128, 8), 1)
    fsel = jnp.where(fi == fj, jnp.tile(fcc.reshape(D, 1), (8, 8)),
                     jnp.zeros((128, 8), jnp.float32))
    inf_ref[...] = jnp.dot(act, fsel, preferred_element_type=jnp.float32)
    regs = REG * (jnp.sqrt(jnp.sum(pe * pe))
                  + jnp.sqrt(jnp.sum(qe * qe))
                  + jnp.sqrt(jnp.sum(re * re)))
    regs_ref[...] = regs.reshape(1, 1)


_tc_tail = pl.pallas_call(
    _tc_tail_body,
    out_shape=(
        jax.ShapeDtypeStruct((ROWS, 8), jnp.float32),
        jax.ShapeDtypeStruct((1, 1), jnp.float32),
    ),
)


def kernel(ps, qs, rs, Pe, Qe, Re, W_w, W_b, FC_w):
    ps1 = ps.astype(jnp.int32)
    qs1 = qs.astype(jnp.int32)
    rs1 = rs.astype(jnp.int32)
    p1, q1, r1 = _sc_gather()(ps1, qs1, rs1, Pe.T, Qe.T, Re.T)
    pe2 = p1[:BATCH * D].reshape(ROWS, 128)
    qe2 = q1[:BATCH * D].reshape(ROWS, 128)
    re2 = r1[:BATCH * D].reshape(ROWS, 128)
    inf2, regs = _tc_tail(pe2, qe2, re2, W_w, W_b.reshape(1, D), FC_w)
    return inf2.reshape(BATCH, 1), regs[0, 0]


# speedup vs baseline: 2.8074x; 2.8074x over previous
import functools

import jax
import jax.numpy as jnp
from jax import lax
from jax.experimental import pallas as pl
from jax.experimental.pallas import tpu as pltpu
from jax.experimental.pallas import tpu_sc as plsc

BATCH = 16384
D = 16
REG = 0.001
NC, NS = 2, 16
NW = NC * NS
BPW = BATCH // NW
G = 16
NG = BPW // G
NGH = NG // 2
ROWS = BATCH // 8


def _sc_gather_body(ps_hbm, qs_hbm, rs_hbm, pt, qt, rt,
                    po, qo, ro, pidx, qidx, ridx, win, outv, sem0, sem1):
    wid = lax.axis_index("s") * NC + lax.axis_index("c")
    base = wid * BPW
    iota16 = lax.iota(jnp.int32, 16)
    pltpu.sync_copy(ps_hbm.at[pl.ds(base, BPW)], pidx)
    pltpu.sync_copy(qs_hbm.at[pl.ds(base, BPW)], qidx)
    pltpu.sync_copy(rs_hbm.at[pl.ds(base, BPW)], ridx)

    for tab, idxv, out1d in ((pt, pidx, po), (qt, qidx, qo), (rt, ridx, ro)):
        def fire(g, buf, sem, tab=tab, idxv=idxv):
            iv = idxv[pl.ds(g * G, G)]
            cbv = (iv >> 7) * 128
            for k in range(G):
                cb = pl.multiple_of(cbv[k], 128)
                pltpu.async_copy(
                    tab.at[:, pl.ds(cb, 128)], win.at[buf * G + k], sem)

        def harvest(g, buf, sem, tab=tab, idxv=idxv, out1d=out1d):
            j0 = g * G
            iv = idxv[pl.ds(j0, G)]
            lanes = iv & 127
            for k in range(G):
                pltpu.make_async_copy(
                    tab.at[:, pl.ds(0, 128)], win.at[buf * G + k], sem).wait()
            for k in range(G):
                colv = lax.broadcast_in_dim(lanes[k], (16,), ())
                vals = plsc.load_gather(win.at[buf * G + k], (iota16, colv))
                outv[pl.ds((j0 + k) * D, D)] = vals

        def body(i, carry):
            g0 = i * 2
            fire(g0 + 1, 1, sem1)
            harvest(g0, 0, sem0)

            @pl.when(i + 1 < NGH)
            def _():
                fire(g0 + 2, 0, sem0)
            harvest(g0 + 1, 1, sem1)
            return carry

        fire(0, 0, sem0)
        lax.fori_loop(0, NGH, body, 0)
        pltpu.sync_copy(outv, out1d.at[pl.ds(base * D, BPW * D)])


@functools.cache
def _sc_gather():
    mesh = plsc.VectorSubcoreMesh(
        core_axis_name="c", subcore_axis_name="s",
        num_cores=NC, num_subcores=NS)
    return pl.kernel(
        _sc_gather_body,
        out_type=[jax.ShapeDtypeStruct((BATCH * D,), jnp.float32)] * 3,
        mesh=mesh,
        compiler_params=pltpu.CompilerParams(needs_layout_passes=False),
        scratch_types=[
            pltpu.VMEM((BPW,), jnp.int32),
            pltpu.VMEM((BPW,), jnp.int32),
            pltpu.VMEM((BPW,), jnp.int32),
            pltpu.VMEM((2 * G, 16, 128), jnp.float32),
            pltpu.VMEM((BPW * D,), jnp.float32),
            pltpu.SemaphoreType.DMA,
            pltpu.SemaphoreType.DMA,
        ],
    )


def _tc_tail_body(pe_ref, qe_ref, re_ref, ww_ref, wb_ref, fcw_ref,
                  inf_ref, regs_ref):
    ww = ww_ref[...]
    wn = jnp.sqrt(jnp.sum(ww * ww, axis=1, keepdims=True))
    wc = ww / jnp.maximum(wn, 1.0)
    fc = fcw_ref[...]
    fn = jnp.sqrt(jnp.sum(fc * fc, axis=1, keepdims=True))
    fcc = fc / jnp.maximum(fn, 1.0)
    pe = pe_ref[...]
    qe = qe_ref[...]
    re = re_ref[...]
    gmf = pe * qe * re
    ri = lax.broadcasted_iota(jnp.int32, (128, 128), 0) // D
    ci = lax.broadcasted_iota(jnp.int32, (128, 128), 1) // D
    blk = ri == ci
    zero = jnp.zeros((128, 128), jnp.float32)
    mp = jnp.where(blk, jnp.tile(wc[:, :D].T, (8, 8)), zero)
    mq = jnp.where(blk, jnp.tile(wc[:, D:2 * D].T, (8, 8)), zero)
    mr = jnp.where(blk, jnp.tile(wc[:, 2 * D:].T, (8, 8)), zero)
    wb_t = jnp.tile(wb_ref[...], (1, 8))
    mlp = (jnp.dot(pe, mp, preferred_element_type=jnp.float32)
           + jnp.dot(qe, mq, preferred_element_type=jnp.float32)
           + jnp.dot(re, mr, preferred_element_type=jnp.float32)
           + wb_t)
    act = jnp.maximum(gmf + mlp, 0.0)
    fi = lax.broadcasted_iota(jnp.int32, (128, 8), 0) // D
    fj = lax.broadcasted_iota(jnp.int32, (128, 8), 1)
    fsel = jnp.where(fi == fj, jnp.tile(fcc.reshape(D, 1), (8, 8)),
                     jnp.zeros((128, 8), jnp.float32))
    inf_ref[...] = jnp.dot(act, fsel, preferred_element_type=jnp.float32)
    regs = REG * (jnp.sqrt(jnp.sum(pe * pe))
                  + jnp.sqrt(jnp.sum(qe * qe))
                  + jnp.sqrt(jnp.sum(re * re)))
    regs_ref[...] = regs.reshape(1, 1)


_tc_tail = pl.pallas_call(
    _tc_tail_body,
    out_shape=(
        jax.ShapeDtypeStruct((ROWS, 8), jnp.float32),
        jax.ShapeDtypeStruct((1, 1), jnp.float32),
    ),
)


def kernel(ps, qs, rs, Pe, Qe, Re, W_w, W_b, FC_w):
    ps1 = ps.astype(jnp.int32)
    qs1 = qs.astype(jnp.int32)
    rs1 = rs.astype(jnp.int32)
    p1, q1, r1 = _sc_gather()(ps1, qs1, rs1, Pe.T, Qe.T, Re.T)
    pe2 = p1.reshape(ROWS, 128)
    qe2 = q1.reshape(ROWS, 128)
    re2 = r1.reshape(ROWS, 128)
    inf2, regs = _tc_tail(pe2, qe2, re2, W_w, W_b.reshape(1, D), FC_w)
    return inf2.reshape(BATCH, 1), regs[0, 0]
